# Initial kernel scaffold; baseline (speedup 1.0000x reference)
import jax
import jax.numpy as jnp
from jax.experimental import pallas as pl


def kernel(idx, symbol_table, concept_table, law_table, pos_table, alpha_logit, beta_logit, W1, b1, W2, b2):
    raise NotImplementedError("write your pallas kernel here")


# speedup vs baseline: 4.8589x; 4.8589x over previous
import functools

import jax
import jax.numpy as jnp
from jax import lax
from jax.experimental import pallas as pl
from jax.experimental.pallas import tpu as pltpu
from jax.experimental.pallas import tpu_sc as plsc


def _fused_table_body(al_ref, be_ref, sym_ref, con_ref, law_ref,
                      w1_ref, b1_ref, w2_ref, b2_ref, out_ref):
    alpha = jax.nn.sigmoid(al_ref[0, 0])
    beta = jax.nn.sigmoid(be_ref[0, 0])
    c = con_ref[:, :]
    h = lax.dot_general(c, w1_ref[:, :], (((1,), (1,)), ((), ())),
                        preferred_element_type=jnp.float32)
    h = jnp.maximum(h + b1_ref[:, :], 0.0)
    cr = lax.dot_general(h, w2_ref[:, :], (((1,), (1,)), ((), ())),
                         preferred_element_type=jnp.float32)
    cr = cr + b2_ref[:, :]
    out_ref[:, :] = sym_ref[:, :] + alpha * cr + beta * law_ref[:, :]


def _make_fused_table(symbol_table, concept_table, law_table,
                      alpha_logit, beta_logit, W1, b1, W2, b2):
    V, C = symbol_table.shape
    RB = 2000
    assert V % RB == 0
    grid = (V // RB,)
    tab_spec = pl.BlockSpec((RB, C), lambda i: (i, 0))
    full = lambda shape: pl.BlockSpec(shape, lambda i: (0, 0))
    return pl.pallas_call(
        _fused_table_body,
        grid=grid,
        in_specs=[full((1, 1)), full((1, 1)),
                  tab_spec, tab_spec, tab_spec,
                  full((C, C)), full((1, C)), full((C, C)), full((1, C))],
        out_specs=tab_spec,
        out_shape=jax.ShapeDtypeStruct((V, C), jnp.float32),
    )(alpha_logit.reshape(1, 1), beta_logit.reshape(1, 1),
      symbol_table, concept_table, law_table,
      W1, b1.reshape(1, C), W2, b2.reshape(1, C))


def _gather_pos(fused, idx2, pos, B, T, C, NW, mesh):
    frames_per_worker = B // NW
    chunks = T // 100

    def body(fused_hbm, idx_hbm, pos_hbm, out_hbm, idx_v, rows_v, pos_v, sem):
        wid = lax.axis_index("c") * mesh.num_subcores + lax.axis_index("s")
        pltpu.sync_copy(pos_hbm, pos_v)

        def frame_body(j, carry):
            f = wid * frames_per_worker + j
            pltpu.sync_copy(idx_hbm.at[pl.ds(f * chunks, chunks)], idx_v)
            cps = []
            for k in range(chunks):
                cps.append(pltpu.async_copy(
                    fused_hbm.at[idx_v.at[k]],
                    rows_v.at[pl.ds(k * 100, 100)], sem))
            for cp in cps:
                cp.wait()

            def row_body(r, carry2):
                for c2 in range(C // 16):
                    sl = pl.ds(c2 * 16, 16)
                    plsc.addupdate(rows_v.at[r, sl], pos_v[r, sl])
                return carry2

            lax.fori_loop(0, T, row_body, 0, unroll=2)
            pltpu.sync_copy(rows_v, out_hbm.at[pl.ds(f * T, T)])
            return carry

        lax.fori_loop(0, frames_per_worker, frame_body, 0)

    k = pl.kernel(
        body,
        out_type=jax.ShapeDtypeStruct((B * T, C), jnp.float32),
        mesh=mesh,
        compiler_params=pltpu.CompilerParams(use_tc_tiling_on_sc=False),
        scratch_types=[
            pltpu.VMEM((chunks, 100), jnp.int32),
            pltpu.VMEM((T, C), jnp.float32),
            pltpu.VMEM((T, C), jnp.float32),
            pltpu.SemaphoreType.DMA,
        ],
    )
    return k(fused, idx2, pos)


def kernel(idx, symbol_table, concept_table, law_table, pos_table,
           alpha_logit, beta_logit, W1, b1, W2, b2):
    B, T = idx.shape
    V, C = symbol_table.shape
    fused = _make_fused_table(symbol_table, concept_table, law_table,
                              alpha_logit, beta_logit, W1, b1, W2, b2)
    mesh = plsc.VectorSubcoreMesh(core_axis_name="c", subcore_axis_name="s")
    NW = mesh.num_cores * mesh.num_subcores
    assert T % 100 == 0 and B % NW == 0 and C % 16 == 0
    idx2 = idx.reshape(B * T // 100, 100)
    pos = pos_table[:T]
    out = _gather_pos(fused, idx2, pos, B, T, C, NW, mesh)
    return out.reshape(B, T, C)


# speedup vs baseline: 5.4539x; 1.1225x over previous
import functools

import jax
import jax.numpy as jnp
from jax import lax
from jax.experimental import pallas as pl
from jax.experimental.pallas import tpu as pltpu
from jax.experimental.pallas import tpu_sc as plsc


def _fused_table_body(al_ref, be_ref, sym_ref, con_ref, law_ref,
                      w1_ref, b1_ref, w2_ref, b2_ref, out_ref):
    alpha = jax.nn.sigmoid(al_ref[0, 0])
    beta = jax.nn.sigmoid(be_ref[0, 0])
    c = con_ref[:, :]
    h = lax.dot_general(c, w1_ref[:, :], (((1,), (1,)), ((), ())),
                        preferred_element_type=jnp.float32)
    h = jnp.maximum(h + b1_ref[:, :], 0.0)
    cr = lax.dot_general(h, w2_ref[:, :], (((1,), (1,)), ((), ())),
                         preferred_element_type=jnp.float32)
    cr = cr + b2_ref[:, :]
    out_ref[:, :] = sym_ref[:, :] + alpha * cr + beta * law_ref[:, :]


def _make_fused_table(symbol_table, concept_table, law_table,
                      alpha_logit, beta_logit, W1, b1, W2, b2):
    V, C = symbol_table.shape
    RB = 2000
    assert V % RB == 0
    grid = (V // RB,)
    tab_spec = pl.BlockSpec((RB, C), lambda i: (i, 0))
    full = lambda shape: pl.BlockSpec(shape, lambda i: (0, 0))
    return pl.pallas_call(
        _fused_table_body,
        grid=grid,
        in_specs=[full((1, 1)), full((1, 1)),
                  tab_spec, tab_spec, tab_spec,
                  full((C, C)), full((1, C)), full((C, C)), full((1, C))],
        out_specs=tab_spec,
        out_shape=jax.ShapeDtypeStruct((V, C), jnp.float32),
    )(alpha_logit.reshape(1, 1), beta_logit.reshape(1, 1),
      symbol_table, concept_table, law_table,
      W1, b1.reshape(1, C), W2, b2.reshape(1, C))


def _gather_pos(fused, idx2, pos, B, T, C, NW, mesh):
    FPW = B // NW
    CH = T // 100
    assert FPW % 2 == 0

    def body(fused_hbm, idx_hbm, pos_hbm, out_hbm,
             idx_a, idx_b, rows_a, rows_b, pos_v,
             sg_a, sg_b, so_a, so_b, si_a, si_b):
        wid = lax.axis_index("c") * mesh.num_subcores + lax.axis_index("s")
        base = wid * FPW
        pltpu.sync_copy(pos_hbm, pos_v)

        idx_ref = [idx_a, idx_b]
        rows_ref = [rows_a, rows_b]
        sg = [sg_a, sg_b]
        so = [so_a, so_b]
        si = [si_a, si_b]

        def start_gather(f, b):
            for k in range(CH):
                pltpu.async_copy(fused_hbm.at[idx_ref[b].at[k]],
                                 rows_ref[b].at[pl.ds(k * 100, 100)], sg[b])

        def wait_gather(b):
            for k in range(CH):
                pltpu.make_async_copy(
                    fused_hbm.at[idx_ref[b].at[0]],
                    rows_ref[b].at[pl.ds(0, 100)], sg[b]).wait()

        def start_idx(f, b):
            pltpu.async_copy(idx_hbm.at[pl.ds(f * CH, CH)], idx_ref[b], si[b])

        def wait_idx(b):
            pltpu.make_async_copy(idx_hbm.at[pl.ds(0, CH)],
                                  idx_ref[b], si[b]).wait()

        def pos_add(b):
            def row_body(r, carry2):
                for c2 in range(C // 16):
                    sl = pl.ds(c2 * 16, 16)
                    plsc.addupdate(rows_ref[b].at[r, sl], pos_v[r, sl])
                return carry2
            lax.fori_loop(0, T, row_body, 0, unroll=2)

        def start_out(f, b):
            pltpu.async_copy(rows_ref[b], out_hbm.at[pl.ds(f * T, T)], so[b])

        def wait_out(b):
            pltpu.make_async_copy(rows_ref[b],
                                  out_hbm.at[pl.ds(0, T)], so[b]).wait()

        pltpu.sync_copy(idx_hbm.at[pl.ds(base * CH, CH)], idx_a)
        start_gather(base, 0)
        start_idx(base + 1, 1)

        def pair_body(jj, carry):
            j = jj * 2

            def half(j, b):
                f = base + j
                nb = 1 - b
                @pl.when(j + 1 < FPW)
                def _():
                    wait_idx(nb)
                    @pl.when(j >= 1)
                    def _():
                        wait_out(nb)
                    start_gather(f + 1, nb)
                wait_gather(b)
                @pl.when(j + 2 < FPW)
                def _():
                    start_idx(f + 2, b)
                pos_add(b)
                start_out(f, b)

            half(j, 0)
            half(j + 1, 1)
            return carry

        lax.fori_loop(0, FPW // 2, pair_body, 0)
        wait_out(0)
        wait_out(1)

    k = pl.kernel(
        body,
        out_type=jax.ShapeDtypeStruct((B * T, C), jnp.float32),
        mesh=mesh,
        compiler_params=pltpu.CompilerParams(use_tc_tiling_on_sc=False),
        scratch_types=[
            pltpu.VMEM((CH, 100), jnp.int32),
            pltpu.VMEM((CH, 100), jnp.int32),
            pltpu.VMEM((T, C), jnp.float32),
            pltpu.VMEM((T, C), jnp.float32),
            pltpu.VMEM((T, C), jnp.float32),
            pltpu.SemaphoreType.DMA,
            pltpu.SemaphoreType.DMA,
            pltpu.SemaphoreType.DMA,
            pltpu.SemaphoreType.DMA,
            pltpu.SemaphoreType.DMA,
            pltpu.SemaphoreType.DMA,
        ],
    )
    return k(fused, idx2, pos)


def kernel(idx, symbol_table, concept_table, law_table, pos_table,
           alpha_logit, beta_logit, W1, b1, W2, b2):
    B, T = idx.shape
    V, C = symbol_table.shape
    fused = _make_fused_table(symbol_table, concept_table, law_table,
                              alpha_logit, beta_logit, W1, b1, W2, b2)
    mesh = plsc.VectorSubcoreMesh(core_axis_name="c", subcore_axis_name="s")
    NW = mesh.num_cores * mesh.num_subcores
    assert T % 100 == 0 and B % NW == 0 and C % 16 == 0
    idx2 = idx.reshape(B * T // 100, 100)
    pos = pos_table[:T]
    out = _gather_pos(fused, idx2, pos, B, T, C, NW, mesh)
    return out.reshape(B, T, C)


# speedup vs baseline: 5.6300x; 1.0323x over previous
import functools

import jax
import jax.numpy as jnp
from jax import lax
from jax.experimental import pallas as pl
from jax.experimental.pallas import tpu as pltpu
from jax.experimental.pallas import tpu_sc as plsc


_NBUF = 4
_NCH = 20


def _fused_table_body(al_ref, be_ref, w1_ref, b1_ref, w2_ref, b2_ref,
                      sym_hbm, con_hbm, law_hbm, out_hbm,
                      sym_v, con_v, law_v, out_v, rsem, wsem):
    V = sym_hbm.shape[0]
    RB = V // _NCH
    alpha = jax.nn.sigmoid(al_ref[0, 0])
    beta = jax.nn.sigmoid(be_ref[0, 0])

    def start_read(c, s):
        sl = pl.ds(c * RB, RB)
        pltpu.make_async_copy(sym_hbm.at[sl, :], sym_v.at[s], rsem.at[s]).start()
        pltpu.make_async_copy(con_hbm.at[sl, :], con_v.at[s], rsem.at[s]).start()
        pltpu.make_async_copy(law_hbm.at[sl, :], law_v.at[s], rsem.at[s]).start()

    def wait_read(c, s):
        sl = pl.ds(c * RB, RB)
        pltpu.make_async_copy(sym_hbm.at[sl, :], sym_v.at[s], rsem.at[s]).wait()
        pltpu.make_async_copy(con_hbm.at[sl, :], con_v.at[s], rsem.at[s]).wait()
        pltpu.make_async_copy(law_hbm.at[sl, :], law_v.at[s], rsem.at[s]).wait()

    def start_write(c, s):
        pltpu.make_async_copy(out_v.at[s], out_hbm.at[pl.ds(c * RB, RB), :],
                              wsem.at[s]).start()

    def wait_write(c, s):
        pltpu.make_async_copy(out_v.at[s], out_hbm.at[pl.ds(c * RB, RB), :],
                              wsem.at[s]).wait()

    for s in range(_NBUF):
        start_read(s, s)

    def round_body(rr, carry):
        c0 = rr * _NBUF

        for s in range(_NBUF):
            c = c0 + s
            wait_read(c, s)

            @pl.when(c >= _NBUF)
            def _():
                wait_write(c - _NBUF, s)

            con = con_v[s]
            h = lax.dot_general(con, w1_ref[:, :], (((1,), (1,)), ((), ())),
                                preferred_element_type=jnp.float32)
            h = jnp.maximum(h + b1_ref[:, :], 0.0)
            cr = lax.dot_general(h, w2_ref[:, :], (((1,), (1,)), ((), ())),
                                 preferred_element_type=jnp.float32)
            cr = cr + b2_ref[:, :]
            out_v[s] = sym_v[s] + alpha * cr + beta * law_v[s]
            start_write(c, s)

            @pl.when(c + _NBUF < _NCH)
            def _():
                start_read(c + _NBUF, s)
        return carry

    lax.fori_loop(0, _NCH // _NBUF, round_body, 0)
    for s in range(_NBUF):
        wait_write(_NCH - _NBUF + s, s)


def _make_fused_table(symbol_table, concept_table, law_table,
                      alpha_logit, beta_logit, W1, b1, W2, b2):
    V, C = symbol_table.shape
    assert V % _NCH == 0 and (V // _NCH) % 8 == 0 and _NCH % _NBUF == 0
    RB = V // _NCH
    small = lambda shape: pl.BlockSpec(shape, lambda: (0,) * len(shape))
    hbm = pl.BlockSpec(memory_space=pl.ANY)
    return pl.pallas_call(
        _fused_table_body,
        in_specs=[small((1, 1)), small((1, 1)),
                  small((C, C)), small((1, C)), small((C, C)), small((1, C)),
                  hbm, hbm, hbm],
        out_specs=hbm,
        out_shape=jax.ShapeDtypeStruct((V, C), jnp.float32),
        scratch_shapes=[
            pltpu.VMEM((_NBUF, RB, C), jnp.float32),
            pltpu.VMEM((_NBUF, RB, C), jnp.float32),
            pltpu.VMEM((_NBUF, RB, C), jnp.float32),
            pltpu.VMEM((_NBUF, RB, C), jnp.float32),
            pltpu.SemaphoreType.DMA((_NBUF,)),
            pltpu.SemaphoreType.DMA((_NBUF,)),
        ],
    )(alpha_logit.reshape(1, 1), beta_logit.reshape(1, 1),
      W1, b1.reshape(1, C), W2, b2.reshape(1, C),
      symbol_table, concept_table, law_table)


def _gather_pos(fused, idx2, pos, B, T, C, NW, mesh):
    FPW = B // NW
    CH = T // 100
    assert FPW % 2 == 0

    def body(fused_hbm, idx_hbm, pos_hbm, out_hbm,
             idx_a, idx_b, rows_a, rows_b, pos_v,
             sg_a, sg_b, so_a, so_b, si_a, si_b):
        wid = lax.axis_index("c") * mesh.num_subcores + lax.axis_index("s")
        base = wid * FPW
        pltpu.sync_copy(pos_hbm, pos_v)

        idx_ref = [idx_a, idx_b]
        rows_ref = [rows_a, rows_b]
        sg = [sg_a, sg_b]
        so = [so_a, so_b]
        si = [si_a, si_b]

        def start_gather(f, b):
            for k in range(CH):
                pltpu.async_copy(fused_hbm.at[idx_ref[b].at[k]],
                                 rows_ref[b].at[pl.ds(k * 100, 100)], sg[b])

        def wait_gather(b):
            for k in range(CH):
                pltpu.make_async_copy(
                    fused_hbm.at[idx_ref[b].at[0]],
                    rows_ref[b].at[pl.ds(0, 100)], sg[b]).wait()

        def start_idx(f, b):
            pltpu.async_copy(idx_hbm.at[pl.ds(f * CH, CH)], idx_ref[b], si[b])

        def wait_idx(b):
            pltpu.make_async_copy(idx_hbm.at[pl.ds(0, CH)],
                                  idx_ref[b], si[b]).wait()

        def pos_add(b):
            def row_body(r, carry2):
                for c2 in range(C // 16):
                    sl = pl.ds(c2 * 16, 16)
                    plsc.addupdate(rows_ref[b].at[r, sl], pos_v[r, sl])
                return carry2
            lax.fori_loop(0, T, row_body, 0, unroll=2)

        def start_out(f, b):
            pltpu.async_copy(rows_ref[b], out_hbm.at[pl.ds(f * T, T)], so[b])

        def wait_out(b):
            pltpu.make_async_copy(rows_ref[b],
                                  out_hbm.at[pl.ds(0, T)], so[b]).wait()

        pltpu.sync_copy(idx_hbm.at[pl.ds(base * CH, CH)], idx_a)
        start_gather(base, 0)
        start_idx(base + 1, 1)

        def pair_body(jj, carry):
            j = jj * 2

            def half(j, b):
                f = base + j
                nb = 1 - b
                @pl.when(j + 1 < FPW)
                def _():
                    wait_idx(nb)
                    @pl.when(j >= 1)
                    def _():
                        wait_out(nb)
                    start_gather(f + 1, nb)
                wait_gather(b)
                @pl.when(j + 2 < FPW)
                def _():
                    start_idx(f + 2, b)
                pos_add(b)
                start_out(f, b)

            half(j, 0)
            half(j + 1, 1)
            return carry

        lax.fori_loop(0, FPW // 2, pair_body, 0)
        wait_out(0)
        wait_out(1)

    k = pl.kernel(
        body,
        out_type=jax.ShapeDtypeStruct((B * T, C), jnp.float32),
        mesh=mesh,
        compiler_params=pltpu.CompilerParams(use_tc_tiling_on_sc=False),
        scratch_types=[
            pltpu.VMEM((CH, 100), jnp.int32),
            pltpu.VMEM((CH, 100), jnp.int32),
            pltpu.VMEM((T, C), jnp.float32),
            pltpu.VMEM((T, C), jnp.float32),
            pltpu.VMEM((T, C), jnp.float32),
            pltpu.SemaphoreType.DMA,
            pltpu.SemaphoreType.DMA,
            pltpu.SemaphoreType.DMA,
            pltpu.SemaphoreType.DMA,
            pltpu.SemaphoreType.DMA,
            pltpu.SemaphoreType.DMA,
        ],
    )
    return k(fused, idx2, pos)


def kernel(idx, symbol_table, concept_table, law_table, pos_table,
           alpha_logit, beta_logit, W1, b1, W2, b2):
    B, T = idx.shape
    V, C = symbol_table.shape
    fused = _make_fused_table(symbol_table, concept_table, law_table,
                              alpha_logit, beta_logit, W1, b1, W2, b2)
    mesh = plsc.VectorSubcoreMesh(core_axis_name="c", subcore_axis_name="s")
    NW = mesh.num_cores * mesh.num_subcores
    assert T % 100 == 0 and B % NW == 0 and C % 16 == 0
    idx2 = idx.reshape(B * T // 100, 100)
    pos = pos_table[:T]
    out = _gather_pos(fused, idx2, pos, B, T, C, NW, mesh)
    return out.reshape(B, T, C)
